# baseline (speedup 1.0000x reference)
import jax
import jax.numpy as jnp
from jax import lax
from jax.experimental import pallas as pl
from jax.experimental.pallas import tpu as pltpu
from jax.experimental.pallas import tpu_sc as plsc

N = 10000
NP = 10112
E = 320000
NCORES = 2
NTILES = 16
NW = NCORES * NTILES
CHUNK = 128
KCH = 80
G = 4
NG = KCH // G
EPT = KCH * CHUNK
EPAD = EPT * NW
RPT = NP // NTILES
HID = 64
RB = 2528
GRID = NP // RB

_sc_mesh = plsc.VectorSubcoreMesh(core_axis_name="c", subcore_axis_name="s")
_sc_params = pltpu.CompilerParams(use_tc_tiling_on_sc=False)


def _agg_body(y_hbm, src_hbm, dst_hbm, zeros_hbm, out_hbm,
              sidx_v, didx_v, bufa_v, bufb_v, acc_sh, gsem, ssem):
    c = lax.axis_index("c")
    s = lax.axis_index("s")
    wid = c * NTILES + s
    pltpu.sync_copy(zeros_hbm.at[pl.ds(s * RPT, RPT)],
                    acc_sh.at[pl.ds(s * RPT, RPT)])
    pltpu.sync_copy(src_hbm.at[wid], sidx_v)
    pltpu.sync_copy(dst_hbm.at[wid], didx_v)
    plsc.subcore_barrier()

    pltpu.async_copy(y_hbm.at[sidx_v.at[0]], bufa_v.at[0], gsem)

    def body(j, carry):
        b = lax.rem(j, 2)
        buf = bufa_v.at[b]
        pltpu.make_async_copy(y_hbm.at[sidx_v.at[j]], buf, gsem).wait()
        pltpu.async_copy(y_hbm.at[sidx_v.at[j + 1]], bufa_v.at[1 - b], gsem)
        pltpu.sync_copy(buf, acc_sh.at[didx_v.at[j]], add=True)
        return carry

    lax.fori_loop(0, KCH - 1, body, 0)
    last = bufa_v.at[(KCH - 1) % 2]
    pltpu.make_async_copy(y_hbm.at[sidx_v.at[KCH - 1]], last, gsem).wait()
    pltpu.sync_copy(last, acc_sh.at[didx_v.at[KCH - 1]], add=True)
    plsc.subcore_barrier()
    pltpu.sync_copy(acc_sh.at[pl.ds(s * RPT, RPT)],
                    out_hbm.at[c, pl.ds(s * RPT, RPT)])


_agg = pl.kernel(
    _agg_body,
    out_type=jax.ShapeDtypeStruct((NCORES, NP, HID), jnp.float32),
    mesh=_sc_mesh,
    compiler_params=_sc_params,
    scratch_types=[
        pltpu.VMEM((KCH, CHUNK), jnp.int32),
        pltpu.VMEM((KCH, CHUNK), jnp.int32),
        pltpu.VMEM((2, CHUNK, HID), jnp.float32),
        pltpu.VMEM((1, CHUNK, HID), jnp.float32),
        pltpu.VMEM_SHARED((NP, HID), jnp.float32),
        pltpu.SemaphoreType.DMA,
        pltpu.SemaphoreType.DMA,
    ],
)


def _deg_body(src_hbm, dst_hbm, ones_hbm, zeros_hbm, out_hbm,
              sidx_v, didx_v, ones_v, acc_out, acc_in):
    c = lax.axis_index("c")
    s = lax.axis_index("s")
    wid = c * NTILES + s
    pltpu.sync_copy(zeros_hbm.at[pl.ds(s * RPT, RPT)],
                    acc_out.at[pl.ds(s * RPT, RPT)])
    pltpu.sync_copy(zeros_hbm.at[pl.ds(s * RPT, RPT)],
                    acc_in.at[pl.ds(s * RPT, RPT)])
    pltpu.sync_copy(src_hbm.at[wid], sidx_v)
    pltpu.sync_copy(dst_hbm.at[wid], didx_v)
    pltpu.sync_copy(ones_hbm, ones_v)
    plsc.subcore_barrier()

    def body(j, carry):
        pltpu.sync_copy(ones_v, acc_out.at[sidx_v.at[j]], add=True)
        pltpu.sync_copy(ones_v, acc_in.at[didx_v.at[j]], add=True)
        return carry

    lax.fori_loop(0, KCH, body, 0)
    plsc.subcore_barrier()
    pltpu.sync_copy(acc_out.at[pl.ds(s * RPT, RPT)],
                    out_hbm.at[c, 0, pl.ds(s * RPT, RPT)])
    pltpu.sync_copy(acc_in.at[pl.ds(s * RPT, RPT)],
                    out_hbm.at[c, 1, pl.ds(s * RPT, RPT)])


_deg = pl.kernel(
    _deg_body,
    out_type=jax.ShapeDtypeStruct((NCORES, 2, NP, 16), jnp.float32),
    mesh=_sc_mesh,
    compiler_params=_sc_params,
    scratch_types=[
        pltpu.VMEM((KCH, CHUNK), jnp.int32),
        pltpu.VMEM((KCH, CHUNK), jnp.int32),
        pltpu.VMEM((CHUNK, 16), jnp.float32),
        pltpu.VMEM_SHARED((NP, 16), jnp.float32),
        pltpu.VMEM_SHARED((NP, 16), jnp.float32),
    ],
)


def _inv_body(do0, do1, di0, di1, m, oinv, iinv):
    mask = m[...]
    od = do0[:, :1] + do1[:, :1]
    idg = di0[:, :1] + di1[:, :1]
    oinv[...] = lax.rsqrt(jnp.maximum(od, 1.0)) * mask
    iinv[...] = lax.rsqrt(jnp.maximum(idg, 1.0)) * mask


_inv = pl.pallas_call(
    _inv_body,
    out_shape=(jax.ShapeDtypeStruct((NP, 1), jnp.float32),
               jax.ShapeDtypeStruct((NP, 1), jnp.float32)),
)


def _l0_body(x_ref, so_ref, w_ref, o_ref):
    o_ref[...] = jnp.dot(x_ref[...] * so_ref[...], w_ref[...],
                         preferred_element_type=jnp.float32)


_l0 = pl.pallas_call(
    _l0_body,
    grid=(GRID,),
    in_specs=[pl.BlockSpec((RB, 128), lambda i: (i, 0)),
              pl.BlockSpec((RB, 1), lambda i: (i, 0)),
              pl.BlockSpec((128, HID), lambda i: (0, 0))],
    out_specs=pl.BlockSpec((RB, HID), lambda i: (i, 0)),
    out_shape=jax.ShapeDtypeStruct((NP, HID), jnp.float32),
)


def _mid_body(p0_ref, p1_ref, si_ref, b_ref, so_ref, w_ref, o_ref):
    x = jnp.maximum((p0_ref[...] + p1_ref[...]) * si_ref[...] + b_ref[...],
                    0.0) * so_ref[...]
    o_ref[...] = jnp.dot(x, w_ref[...], preferred_element_type=jnp.float32)


_mid = pl.pallas_call(
    _mid_body,
    grid=(GRID,),
    in_specs=[pl.BlockSpec((RB, HID), lambda i: (i, 0)),
              pl.BlockSpec((RB, HID), lambda i: (i, 0)),
              pl.BlockSpec((RB, 1), lambda i: (i, 0)),
              pl.BlockSpec((1, HID), lambda i: (0, 0)),
              pl.BlockSpec((RB, 1), lambda i: (i, 0)),
              pl.BlockSpec((HID, HID), lambda i: (0, 0))],
    out_specs=pl.BlockSpec((RB, HID), lambda i: (i, 0)),
    out_shape=jax.ShapeDtypeStruct((NP, HID), jnp.float32),
)


def _last_body(p0_ref, p1_ref, si_ref, w_ref, b_ref, o_ref):
    x = (p0_ref[...] + p1_ref[...]) * si_ref[...]
    o_ref[...] = jax.nn.sigmoid(
        jnp.dot(x, w_ref[...], preferred_element_type=jnp.float32) + b_ref[...])


_last = pl.pallas_call(
    _last_body,
    grid=(GRID,),
    in_specs=[pl.BlockSpec((RB, HID), lambda i: (i, 0)),
              pl.BlockSpec((RB, HID), lambda i: (i, 0)),
              pl.BlockSpec((RB, 1), lambda i: (i, 0)),
              pl.BlockSpec((HID, 128), lambda i: (0, 0)),
              pl.BlockSpec((1, 128), lambda i: (0, 0))],
    out_specs=pl.BlockSpec((RB, 128), lambda i: (i, 0)),
    out_shape=jax.ShapeDtypeStruct((NP, 128), jnp.float32),
)


def kernel(features, edge_index, Ws, bs):
    pad = jnp.full((EPAD - E,), N, jnp.int32)
    src3 = jnp.concatenate([edge_index[0], pad]).reshape(NW, KCH, CHUNK)
    dst3 = jnp.concatenate([edge_index[1], pad]).reshape(NW, KCH, CHUNK)
    feats_p = jnp.pad(features, ((0, NP - N), (0, 0)))
    zeros64 = jnp.zeros((NP, HID), jnp.float32)
    zeros16 = jnp.zeros((NP, 16), jnp.float32)
    ones16 = jnp.ones((CHUNK, 16), jnp.float32)
    mask = jnp.pad(jnp.ones((N, 1), jnp.float32), ((0, NP - N), (0, 0)))

    deg = _deg(src3, dst3, ones16, zeros16)
    so, si = _inv(deg[0, 0], deg[1, 0], deg[0, 1], deg[1, 1], mask)

    y = _l0(feats_p, so, Ws[0])
    for i in range(1, 7):
        p = _agg(y, src3, dst3, zeros64)
        y = _mid(p[0], p[1], si, bs[i - 1].reshape(1, HID), so, Ws[i])
    p = _agg(y, src3, dst3, zeros64)
    h7 = _mid(p[0], p[1], si, bs[6].reshape(1, HID), so,
              jnp.eye(HID, dtype=jnp.float32))
    q = _agg(h7, src3, dst3, zeros64)
    out = _last(q[0], q[1], si, Ws[7], bs[7].reshape(1, 128))
    return out[:N]


# speedup vs baseline: 1.0122x; 1.0122x over previous
import jax
import jax.numpy as jnp
from jax import lax
from jax.experimental import pallas as pl
from jax.experimental.pallas import tpu as pltpu
from jax.experimental.pallas import tpu_sc as plsc

N = 10000
NP = 10112
E = 320000
NCORES = 2
NTILES = 16
NW = NCORES * NTILES
CHUNK = 512
KCH = 20
EPT = KCH * CHUNK
EPAD = EPT * NW
RPT = NP // NTILES
HID = 64
RB = 2528
GRID = NP // RB

_sc_mesh = plsc.VectorSubcoreMesh(core_axis_name="c", subcore_axis_name="s")
_sc_params = pltpu.CompilerParams(use_tc_tiling_on_sc=False)


def _agg_body(y_hbm, src_hbm, dst_hbm, zeros_hbm, out_hbm,
              sidx_v, didx_v, bufa_v, acc_sh, gsem):
    c = lax.axis_index("c")
    s = lax.axis_index("s")
    wid = c * NTILES + s
    pltpu.sync_copy(zeros_hbm.at[pl.ds(s * RPT, RPT)],
                    acc_sh.at[pl.ds(s * RPT, RPT)])
    pltpu.sync_copy(src_hbm.at[wid], sidx_v)
    pltpu.sync_copy(dst_hbm.at[wid], didx_v)
    plsc.subcore_barrier()

    def body(j, carry):
        pltpu.async_copy(y_hbm.at[sidx_v.at[j]], bufa_v, gsem).wait()
        pltpu.sync_copy(bufa_v, acc_sh.at[didx_v.at[j]], add=True)
        return carry

    lax.fori_loop(0, KCH, body, 0)
    plsc.subcore_barrier()
    pltpu.sync_copy(acc_sh.at[pl.ds(s * RPT, RPT)],
                    out_hbm.at[c, pl.ds(s * RPT, RPT)])


_agg = pl.kernel(
    _agg_body,
    out_type=jax.ShapeDtypeStruct((NCORES, NP, HID), jnp.float32),
    mesh=_sc_mesh,
    compiler_params=_sc_params,
    scratch_types=[
        pltpu.VMEM((KCH, CHUNK), jnp.int32),
        pltpu.VMEM((KCH, CHUNK), jnp.int32),
        pltpu.VMEM((CHUNK, HID), jnp.float32),
        pltpu.VMEM_SHARED((NP, HID), jnp.float32),
        pltpu.SemaphoreType.DMA,
    ],
)


def _deg_body(src_hbm, dst_hbm, ones_hbm, zeros_hbm, out_hbm,
              sidx_v, didx_v, ones_v, acc_out, acc_in):
    c = lax.axis_index("c")
    s = lax.axis_index("s")
    wid = c * NTILES + s
    pltpu.sync_copy(zeros_hbm.at[pl.ds(s * RPT, RPT)],
                    acc_out.at[pl.ds(s * RPT, RPT)])
    pltpu.sync_copy(zeros_hbm.at[pl.ds(s * RPT, RPT)],
                    acc_in.at[pl.ds(s * RPT, RPT)])
    pltpu.sync_copy(src_hbm.at[wid], sidx_v)
    pltpu.sync_copy(dst_hbm.at[wid], didx_v)
    pltpu.sync_copy(ones_hbm, ones_v)
    plsc.subcore_barrier()

    def body(j, carry):
        pltpu.sync_copy(ones_v, acc_out.at[sidx_v.at[j]], add=True)
        pltpu.sync_copy(ones_v, acc_in.at[didx_v.at[j]], add=True)
        return carry

    lax.fori_loop(0, KCH, body, 0)
    plsc.subcore_barrier()
    pltpu.sync_copy(acc_out.at[pl.ds(s * RPT, RPT)],
                    out_hbm.at[c, 0, pl.ds(s * RPT, RPT)])
    pltpu.sync_copy(acc_in.at[pl.ds(s * RPT, RPT)],
                    out_hbm.at[c, 1, pl.ds(s * RPT, RPT)])


_deg = pl.kernel(
    _deg_body,
    out_type=jax.ShapeDtypeStruct((NCORES, 2, NP, 16), jnp.float32),
    mesh=_sc_mesh,
    compiler_params=_sc_params,
    scratch_types=[
        pltpu.VMEM((KCH, CHUNK), jnp.int32),
        pltpu.VMEM((KCH, CHUNK), jnp.int32),
        pltpu.VMEM((CHUNK, 16), jnp.float32),
        pltpu.VMEM_SHARED((NP, 16), jnp.float32),
        pltpu.VMEM_SHARED((NP, 16), jnp.float32),
    ],
)


def _inv_body(do0, do1, di0, di1, m, oinv, iinv):
    mask = m[...]
    od = do0[:, :1] + do1[:, :1]
    idg = di0[:, :1] + di1[:, :1]
    oinv[...] = lax.rsqrt(jnp.maximum(od, 1.0)) * mask
    iinv[...] = lax.rsqrt(jnp.maximum(idg, 1.0)) * mask


_inv = pl.pallas_call(
    _inv_body,
    out_shape=(jax.ShapeDtypeStruct((NP, 1), jnp.float32),
               jax.ShapeDtypeStruct((NP, 1), jnp.float32)),
)


def _l0_body(x_ref, so_ref, w_ref, o_ref):
    o_ref[...] = jnp.dot(x_ref[...] * so_ref[...], w_ref[...],
                         preferred_element_type=jnp.float32)


_l0 = pl.pallas_call(
    _l0_body,
    grid=(GRID,),
    in_specs=[pl.BlockSpec((RB, 128), lambda i: (i, 0)),
              pl.BlockSpec((RB, 1), lambda i: (i, 0)),
              pl.BlockSpec((128, HID), lambda i: (0, 0))],
    out_specs=pl.BlockSpec((RB, HID), lambda i: (i, 0)),
    out_shape=jax.ShapeDtypeStruct((NP, HID), jnp.float32),
)


def _mid_body(p0_ref, p1_ref, si_ref, b_ref, so_ref, w_ref, o_ref):
    x = jnp.maximum((p0_ref[...] + p1_ref[...]) * si_ref[...] + b_ref[...],
                    0.0) * so_ref[...]
    o_ref[...] = jnp.dot(x, w_ref[...], preferred_element_type=jnp.float32)


_mid = pl.pallas_call(
    _mid_body,
    grid=(GRID,),
    in_specs=[pl.BlockSpec((RB, HID), lambda i: (i, 0)),
              pl.BlockSpec((RB, HID), lambda i: (i, 0)),
              pl.BlockSpec((RB, 1), lambda i: (i, 0)),
              pl.BlockSpec((1, HID), lambda i: (0, 0)),
              pl.BlockSpec((RB, 1), lambda i: (i, 0)),
              pl.BlockSpec((HID, HID), lambda i: (0, 0))],
    out_specs=pl.BlockSpec((RB, HID), lambda i: (i, 0)),
    out_shape=jax.ShapeDtypeStruct((NP, HID), jnp.float32),
)


def _last_body(p0_ref, p1_ref, si_ref, w_ref, b_ref, o_ref):
    x = (p0_ref[...] + p1_ref[...]) * si_ref[...]
    o_ref[...] = jax.nn.sigmoid(
        jnp.dot(x, w_ref[...], preferred_element_type=jnp.float32) + b_ref[...])


_last = pl.pallas_call(
    _last_body,
    grid=(GRID,),
    in_specs=[pl.BlockSpec((RB, HID), lambda i: (i, 0)),
              pl.BlockSpec((RB, HID), lambda i: (i, 0)),
              pl.BlockSpec((RB, 1), lambda i: (i, 0)),
              pl.BlockSpec((HID, 128), lambda i: (0, 0)),
              pl.BlockSpec((1, 128), lambda i: (0, 0))],
    out_specs=pl.BlockSpec((RB, 128), lambda i: (i, 0)),
    out_shape=jax.ShapeDtypeStruct((NP, 128), jnp.float32),
)


def kernel(features, edge_index, Ws, bs):
    pad = jnp.full((EPAD - E,), N, jnp.int32)
    src3 = jnp.concatenate([edge_index[0], pad]).reshape(NW, KCH, CHUNK)
    dst3 = jnp.concatenate([edge_index[1], pad]).reshape(NW, KCH, CHUNK)
    feats_p = jnp.pad(features, ((0, NP - N), (0, 0)))
    zeros64 = jnp.zeros((NP, HID), jnp.float32)
    zeros16 = jnp.zeros((NP, 16), jnp.float32)
    ones16 = jnp.ones((CHUNK, 16), jnp.float32)
    mask = jnp.pad(jnp.ones((N, 1), jnp.float32), ((0, NP - N), (0, 0)))

    deg = _deg(src3, dst3, ones16, zeros16)
    so, si = _inv(deg[0, 0], deg[1, 0], deg[0, 1], deg[1, 1], mask)

    y = _l0(feats_p, so, Ws[0])
    for i in range(1, 7):
        p = _agg(y, src3, dst3, zeros64)
        y = _mid(p[0], p[1], si, bs[i - 1].reshape(1, HID), so, Ws[i])
    p = _agg(y, src3, dst3, zeros64)
    h7 = _mid(p[0], p[1], si, bs[6].reshape(1, HID), so,
              jnp.eye(HID, dtype=jnp.float32))
    q = _agg(h7, src3, dst3, zeros64)
    out = _last(q[0], q[1], si, Ws[7], bs[7].reshape(1, 128))
    return out[:N]


# speedup vs baseline: 1.9074x; 1.8844x over previous
import jax
import jax.numpy as jnp
from jax import lax
from jax.experimental import pallas as pl
from jax.experimental.pallas import tpu as pltpu
from jax.experimental.pallas import tpu_sc as plsc

N = 10000
NP = 10112
E = 320000
NCORES = 2
NTILES = 16
NW = NCORES * NTILES
CHUNK = 128
KCH = 80
EPT = KCH * CHUNK
EPAD = EPT * NW
RPT = NP // NTILES
HID = 64
RB = 2528
GRID = NP // RB

_sc_mesh = plsc.VectorSubcoreMesh(core_axis_name="c", subcore_axis_name="s")
_sc_params = pltpu.CompilerParams(use_tc_tiling_on_sc=False)


def _agg_body(y_hbm, src_hbm, dst_hbm, zeros_hbm, out_hbm,
              sidx_v, didx_v, bufa_v, acc_sh, y_sh, gsem):
    c = lax.axis_index("c")
    s = lax.axis_index("s")
    wid = c * NTILES + s
    pltpu.sync_copy(zeros_hbm.at[pl.ds(s * RPT, RPT)],
                    acc_sh.at[pl.ds(s * RPT, RPT)])
    pltpu.sync_copy(y_hbm.at[pl.ds(s * RPT, RPT)],
                    y_sh.at[pl.ds(s * RPT, RPT)])
    pltpu.sync_copy(src_hbm.at[wid], sidx_v)
    pltpu.sync_copy(dst_hbm.at[wid], didx_v)
    plsc.subcore_barrier()

    def body(j, carry):
        pltpu.async_copy(y_sh.at[sidx_v.at[j]], bufa_v, gsem).wait()
        pltpu.sync_copy(bufa_v, acc_sh.at[didx_v.at[j]], add=True)
        return carry

    lax.fori_loop(0, KCH, body, 0)
    plsc.subcore_barrier()
    pltpu.sync_copy(acc_sh.at[pl.ds(s * RPT, RPT)],
                    out_hbm.at[c, pl.ds(s * RPT, RPT)])


_agg = pl.kernel(
    _agg_body,
    out_type=jax.ShapeDtypeStruct((NCORES, NP, HID), jnp.float32),
    mesh=_sc_mesh,
    compiler_params=_sc_params,
    scratch_types=[
        pltpu.VMEM((KCH, CHUNK), jnp.int32),
        pltpu.VMEM((KCH, CHUNK), jnp.int32),
        pltpu.VMEM((CHUNK, HID), jnp.float32),
        pltpu.VMEM_SHARED((NP, HID), jnp.float32),
        pltpu.VMEM_SHARED((NP, HID), jnp.float32),
        pltpu.SemaphoreType.DMA,
    ],
)


def _deg_body(src_hbm, dst_hbm, ones_hbm, zeros_hbm, out_hbm,
              sidx_v, didx_v, ones_v, acc_out, acc_in):
    c = lax.axis_index("c")
    s = lax.axis_index("s")
    wid = c * NTILES + s
    pltpu.sync_copy(zeros_hbm.at[pl.ds(s * RPT, RPT)],
                    acc_out.at[pl.ds(s * RPT, RPT)])
    pltpu.sync_copy(zeros_hbm.at[pl.ds(s * RPT, RPT)],
                    acc_in.at[pl.ds(s * RPT, RPT)])
    pltpu.sync_copy(src_hbm.at[wid], sidx_v)
    pltpu.sync_copy(dst_hbm.at[wid], didx_v)
    pltpu.sync_copy(ones_hbm, ones_v)
    plsc.subcore_barrier()

    def body(j, carry):
        pltpu.sync_copy(ones_v, acc_out.at[sidx_v.at[j]], add=True)
        pltpu.sync_copy(ones_v, acc_in.at[didx_v.at[j]], add=True)
        return carry

    lax.fori_loop(0, KCH, body, 0)
    plsc.subcore_barrier()
    pltpu.sync_copy(acc_out.at[pl.ds(s * RPT, RPT)],
                    out_hbm.at[c, 0, pl.ds(s * RPT, RPT)])
    pltpu.sync_copy(acc_in.at[pl.ds(s * RPT, RPT)],
                    out_hbm.at[c, 1, pl.ds(s * RPT, RPT)])


_deg = pl.kernel(
    _deg_body,
    out_type=jax.ShapeDtypeStruct((NCORES, 2, NP, 16), jnp.float32),
    mesh=_sc_mesh,
    compiler_params=_sc_params,
    scratch_types=[
        pltpu.VMEM((KCH, CHUNK), jnp.int32),
        pltpu.VMEM((KCH, CHUNK), jnp.int32),
        pltpu.VMEM((CHUNK, 16), jnp.float32),
        pltpu.VMEM_SHARED((NP, 16), jnp.float32),
        pltpu.VMEM_SHARED((NP, 16), jnp.float32),
    ],
)


def _inv_body(do0, do1, di0, di1, m, oinv, iinv):
    mask = m[...]
    od = do0[:, :1] + do1[:, :1]
    idg = di0[:, :1] + di1[:, :1]
    oinv[...] = lax.rsqrt(jnp.maximum(od, 1.0)) * mask
    iinv[...] = lax.rsqrt(jnp.maximum(idg, 1.0)) * mask


_inv = pl.pallas_call(
    _inv_body,
    out_shape=(jax.ShapeDtypeStruct((NP, 1), jnp.float32),
               jax.ShapeDtypeStruct((NP, 1), jnp.float32)),
)


def _l0_body(x_ref, so_ref, w_ref, o_ref):
    o_ref[...] = jnp.dot(x_ref[...] * so_ref[...], w_ref[...],
                         preferred_element_type=jnp.float32)


_l0 = pl.pallas_call(
    _l0_body,
    grid=(GRID,),
    in_specs=[pl.BlockSpec((RB, 128), lambda i: (i, 0)),
              pl.BlockSpec((RB, 1), lambda i: (i, 0)),
              pl.BlockSpec((128, HID), lambda i: (0, 0))],
    out_specs=pl.BlockSpec((RB, HID), lambda i: (i, 0)),
    out_shape=jax.ShapeDtypeStruct((NP, HID), jnp.float32),
)


def _mid_body(p0_ref, p1_ref, si_ref, b_ref, so_ref, w_ref, o_ref):
    x = jnp.maximum((p0_ref[...] + p1_ref[...]) * si_ref[...] + b_ref[...],
                    0.0) * so_ref[...]
    o_ref[...] = jnp.dot(x, w_ref[...], preferred_element_type=jnp.float32)


_mid = pl.pallas_call(
    _mid_body,
    grid=(GRID,),
    in_specs=[pl.BlockSpec((RB, HID), lambda i: (i, 0)),
              pl.BlockSpec((RB, HID), lambda i: (i, 0)),
              pl.BlockSpec((RB, 1), lambda i: (i, 0)),
              pl.BlockSpec((1, HID), lambda i: (0, 0)),
              pl.BlockSpec((RB, 1), lambda i: (i, 0)),
              pl.BlockSpec((HID, HID), lambda i: (0, 0))],
    out_specs=pl.BlockSpec((RB, HID), lambda i: (i, 0)),
    out_shape=jax.ShapeDtypeStruct((NP, HID), jnp.float32),
)


def _last_body(p0_ref, p1_ref, si_ref, w_ref, b_ref, o_ref):
    x = (p0_ref[...] + p1_ref[...]) * si_ref[...]
    o_ref[...] = jax.nn.sigmoid(
        jnp.dot(x, w_ref[...], preferred_element_type=jnp.float32) + b_ref[...])


_last = pl.pallas_call(
    _last_body,
    grid=(GRID,),
    in_specs=[pl.BlockSpec((RB, HID), lambda i: (i, 0)),
              pl.BlockSpec((RB, HID), lambda i: (i, 0)),
              pl.BlockSpec((RB, 1), lambda i: (i, 0)),
              pl.BlockSpec((HID, 128), lambda i: (0, 0)),
              pl.BlockSpec((1, 128), lambda i: (0, 0))],
    out_specs=pl.BlockSpec((RB, 128), lambda i: (i, 0)),
    out_shape=jax.ShapeDtypeStruct((NP, 128), jnp.float32),
)


def kernel(features, edge_index, Ws, bs):
    pad = jnp.full((EPAD - E,), N, jnp.int32)
    src3 = jnp.concatenate([edge_index[0], pad]).reshape(NW, KCH, CHUNK)
    dst3 = jnp.concatenate([edge_index[1], pad]).reshape(NW, KCH, CHUNK)
    feats_p = jnp.pad(features, ((0, NP - N), (0, 0)))
    zeros64 = jnp.zeros((NP, HID), jnp.float32)
    zeros16 = jnp.zeros((NP, 16), jnp.float32)
    ones16 = jnp.ones((CHUNK, 16), jnp.float32)
    mask = jnp.pad(jnp.ones((N, 1), jnp.float32), ((0, NP - N), (0, 0)))

    deg = _deg(src3, dst3, ones16, zeros16)
    so, si = _inv(deg[0, 0], deg[1, 0], deg[0, 1], deg[1, 1], mask)

    y = _l0(feats_p, so, Ws[0])
    for i in range(1, 7):
        p = _agg(y, src3, dst3, zeros64)
        y = _mid(p[0], p[1], si, bs[i - 1].reshape(1, HID), so, Ws[i])
    p = _agg(y, src3, dst3, zeros64)
    h7 = _mid(p[0], p[1], si, bs[6].reshape(1, HID), so,
              jnp.eye(HID, dtype=jnp.float32))
    q = _agg(h7, src3, dst3, zeros64)
    out = _last(q[0], q[1], si, Ws[7], bs[7].reshape(1, 128))
    return out[:N]


# speedup vs baseline: 1.9153x; 1.0041x over previous
import jax
import jax.numpy as jnp
from jax import lax
from jax.experimental import pallas as pl
from jax.experimental.pallas import tpu as pltpu
from jax.experimental.pallas import tpu_sc as plsc

N = 10000
NP = 10112
E = 320000
NCORES = 2
NTILES = 16
NW = NCORES * NTILES
CHUNK = 128
KCH = 80
DCH = 512
DKCH = 20
EPT = KCH * CHUNK
EPAD = EPT * NW
RPT = NP // NTILES
HID = 64
RB = 2528
GRID = NP // RB

_sc_mesh = plsc.VectorSubcoreMesh(core_axis_name="c", subcore_axis_name="s")
_sc_params = pltpu.CompilerParams(use_tc_tiling_on_sc=False)


def _agg_body(y_hbm, src_hbm, dst_hbm, zeros_hbm, out_hbm,
              sidx_v, didx_v, bufa_v, acc_sh, y_sh, gsem):
    c = lax.axis_index("c")
    s = lax.axis_index("s")
    wid = c * NTILES + s
    pltpu.sync_copy(zeros_hbm.at[pl.ds(s * RPT, RPT)],
                    acc_sh.at[pl.ds(s * RPT, RPT)])
    pltpu.sync_copy(y_hbm.at[pl.ds(s * RPT, RPT)],
                    y_sh.at[pl.ds(s * RPT, RPT)])
    pltpu.sync_copy(src_hbm.at[wid], sidx_v)
    pltpu.sync_copy(dst_hbm.at[wid], didx_v)
    plsc.subcore_barrier()

    def body(j, carry):
        pltpu.async_copy(y_sh.at[sidx_v.at[j]], bufa_v, gsem).wait()
        pltpu.sync_copy(bufa_v, acc_sh.at[didx_v.at[j]], add=True)
        return carry

    lax.fori_loop(0, KCH, body, 0)
    plsc.subcore_barrier()
    pltpu.sync_copy(acc_sh.at[pl.ds(s * RPT, RPT)],
                    out_hbm.at[c, pl.ds(s * RPT, RPT)])


_agg = pl.kernel(
    _agg_body,
    out_type=jax.ShapeDtypeStruct((NCORES, NP, HID), jnp.float32),
    mesh=_sc_mesh,
    compiler_params=_sc_params,
    scratch_types=[
        pltpu.VMEM((KCH, CHUNK), jnp.int32),
        pltpu.VMEM((KCH, CHUNK), jnp.int32),
        pltpu.VMEM((CHUNK, HID), jnp.float32),
        pltpu.VMEM_SHARED((NP, HID), jnp.float32),
        pltpu.VMEM_SHARED((NP, HID), jnp.float32),
        pltpu.SemaphoreType.DMA,
    ],
)


def _deg_body(src_hbm, dst_hbm, ones_hbm, zeros_hbm, out_hbm,
              sidx_v, didx_v, ones_v, acc_out, acc_in):
    c = lax.axis_index("c")
    s = lax.axis_index("s")
    wid = c * NTILES + s
    pltpu.sync_copy(zeros_hbm.at[pl.ds(s * RPT, RPT)],
                    acc_out.at[pl.ds(s * RPT, RPT)])
    pltpu.sync_copy(zeros_hbm.at[pl.ds(s * RPT, RPT)],
                    acc_in.at[pl.ds(s * RPT, RPT)])
    pltpu.sync_copy(src_hbm.at[wid], sidx_v)
    pltpu.sync_copy(dst_hbm.at[wid], didx_v)
    pltpu.sync_copy(ones_hbm, ones_v)
    plsc.subcore_barrier()

    def body(j, carry):
        pltpu.sync_copy(ones_v, acc_out.at[sidx_v.at[j]], add=True)
        pltpu.sync_copy(ones_v, acc_in.at[didx_v.at[j]], add=True)
        return carry

    lax.fori_loop(0, DKCH, body, 0)
    plsc.subcore_barrier()
    pltpu.sync_copy(acc_out.at[pl.ds(s * RPT, RPT)],
                    out_hbm.at[c, 0, pl.ds(s * RPT, RPT)])
    pltpu.sync_copy(acc_in.at[pl.ds(s * RPT, RPT)],
                    out_hbm.at[c, 1, pl.ds(s * RPT, RPT)])


_deg = pl.kernel(
    _deg_body,
    out_type=jax.ShapeDtypeStruct((NCORES, 2, NP, 16), jnp.float32),
    mesh=_sc_mesh,
    compiler_params=_sc_params,
    scratch_types=[
        pltpu.VMEM((DKCH, DCH), jnp.int32),
        pltpu.VMEM((DKCH, DCH), jnp.int32),
        pltpu.VMEM((DCH, 16), jnp.float32),
        pltpu.VMEM_SHARED((NP, 16), jnp.float32),
        pltpu.VMEM_SHARED((NP, 16), jnp.float32),
    ],
)


def _l0_body(x_ref, w_ref, d00, d01, d10, d11, m_ref, y_ref, so_ref, si_ref):
    mask = m_ref[...]
    so = lax.rsqrt(jnp.maximum(d00[:, :1] + d01[:, :1], 1.0)) * mask
    si = lax.rsqrt(jnp.maximum(d10[:, :1] + d11[:, :1], 1.0)) * mask
    so_ref[...] = so
    si_ref[...] = si
    y_ref[...] = jnp.dot(x_ref[...] * so, w_ref[...],
                         preferred_element_type=jnp.float32)


_l0 = pl.pallas_call(
    _l0_body,
    grid=(GRID,),
    in_specs=[pl.BlockSpec((RB, 128), lambda i: (i, 0)),
              pl.BlockSpec((128, HID), lambda i: (0, 0)),
              pl.BlockSpec((RB, 16), lambda i: (i, 0)),
              pl.BlockSpec((RB, 16), lambda i: (i, 0)),
              pl.BlockSpec((RB, 16), lambda i: (i, 0)),
              pl.BlockSpec((RB, 16), lambda i: (i, 0)),
              pl.BlockSpec((RB, 1), lambda i: (i, 0))],
    out_specs=(pl.BlockSpec((RB, HID), lambda i: (i, 0)),
               pl.BlockSpec((RB, 1), lambda i: (i, 0)),
               pl.BlockSpec((RB, 1), lambda i: (i, 0))),
    out_shape=(jax.ShapeDtypeStruct((NP, HID), jnp.float32),
               jax.ShapeDtypeStruct((NP, 1), jnp.float32),
               jax.ShapeDtypeStruct((NP, 1), jnp.float32)),
)


def _mid_body(p0_ref, p1_ref, si_ref, b_ref, so_ref, w_ref, o_ref):
    x = jnp.maximum((p0_ref[...] + p1_ref[...]) * si_ref[...] + b_ref[...],
                    0.0) * so_ref[...]
    o_ref[...] = jnp.dot(x, w_ref[...], preferred_element_type=jnp.float32)


_mid = pl.pallas_call(
    _mid_body,
    grid=(GRID,),
    in_specs=[pl.BlockSpec((RB, HID), lambda i: (i, 0)),
              pl.BlockSpec((RB, HID), lambda i: (i, 0)),
              pl.BlockSpec((RB, 1), lambda i: (i, 0)),
              pl.BlockSpec((1, HID), lambda i: (0, 0)),
              pl.BlockSpec((RB, 1), lambda i: (i, 0)),
              pl.BlockSpec((HID, HID), lambda i: (0, 0))],
    out_specs=pl.BlockSpec((RB, HID), lambda i: (i, 0)),
    out_shape=jax.ShapeDtypeStruct((NP, HID), jnp.float32),
)


def _last_body(p0_ref, p1_ref, si_ref, w_ref, b_ref, o_ref):
    x = (p0_ref[...] + p1_ref[...]) * si_ref[...]
    o_ref[...] = jax.nn.sigmoid(
        jnp.dot(x, w_ref[...], preferred_element_type=jnp.float32) + b_ref[...])


_last = pl.pallas_call(
    _last_body,
    grid=(GRID,),
    in_specs=[pl.BlockSpec((RB, HID), lambda i: (i, 0)),
              pl.BlockSpec((RB, HID), lambda i: (i, 0)),
              pl.BlockSpec((RB, 1), lambda i: (i, 0)),
              pl.BlockSpec((HID, 128), lambda i: (0, 0)),
              pl.BlockSpec((1, 128), lambda i: (0, 0))],
    out_specs=pl.BlockSpec((RB, 128), lambda i: (i, 0)),
    out_shape=jax.ShapeDtypeStruct((NP, 128), jnp.float32),
)


def kernel(features, edge_index, Ws, bs):
    pad = jnp.full((EPAD - E,), N, jnp.int32)
    src3 = jnp.concatenate([edge_index[0], pad]).reshape(NW, KCH, CHUNK)
    dst3 = jnp.concatenate([edge_index[1], pad]).reshape(NW, KCH, CHUNK)
    feats_p = jnp.pad(features, ((0, NP - N), (0, 0)))
    zeros64 = jnp.zeros((NP, HID), jnp.float32)
    zeros16 = jnp.zeros((NP, 16), jnp.float32)
    ones16 = jnp.ones((DCH, 16), jnp.float32)
    mask = jnp.pad(jnp.ones((N, 1), jnp.float32), ((0, NP - N), (0, 0)))

    deg = _deg(src3.reshape(NW, DKCH, DCH), dst3.reshape(NW, DKCH, DCH),
               ones16, zeros16)
    y, so, si = _l0(feats_p, Ws[0], deg[0, 0], deg[1, 0], deg[0, 1],
                    deg[1, 1], mask)
    for i in range(1, 7):
        p = _agg(y, src3, dst3, zeros64)
        y = _mid(p[0], p[1], si, bs[i - 1].reshape(1, HID), so, Ws[i])
    p = _agg(y, src3, dst3, zeros64)
    h7 = _mid(p[0], p[1], si, bs[6].reshape(1, HID), so,
              jnp.eye(HID, dtype=jnp.float32))
    q = _agg(h7, src3, dst3, zeros64)
    out = _last(q[0], q[1], si, Ws[7], bs[7].reshape(1, 128))
    return out[:N]


# speedup vs baseline: 2.3762x; 1.2407x over previous
import jax
import jax.numpy as jnp
from jax import lax
from jax.experimental import pallas as pl
from jax.experimental.pallas import tpu as pltpu
from jax.experimental.pallas import tpu_sc as plsc

N = 10000
NP = 10112
E = 320000
NCORES = 2
NTILES = 16
NW = NCORES * NTILES
CHUNK = 128
KCH = 80
DCH = 512
DKCH = 20
EPT = KCH * CHUNK
EPAD = EPT * NW
RPT = NP // NTILES
HID = 64
RB = 2528
GRID = NP // RB

_sc_mesh = plsc.VectorSubcoreMesh(core_axis_name="c", subcore_axis_name="s")
_sc_params = pltpu.CompilerParams(use_tc_tiling_on_sc=False)


def _agg_body(y_hbm, src_hbm, dst_hbm, zeros_hbm, out_hbm,
              sidx_v, didx_v, bufa_v, acc_sh, y_sh, gsem):
    c = lax.axis_index("c")
    s = lax.axis_index("s")
    wid = c * NTILES + s
    pltpu.sync_copy(zeros_hbm.at[pl.ds(s * RPT, RPT)],
                    acc_sh.at[pl.ds(s * RPT, RPT)])
    pltpu.sync_copy(y_hbm.at[pl.ds(s * RPT, RPT)],
                    y_sh.at[pl.ds(s * RPT, RPT)])
    pltpu.sync_copy(src_hbm.at[wid], sidx_v)
    pltpu.sync_copy(dst_hbm.at[wid], didx_v)
    plsc.subcore_barrier()

    pltpu.async_copy(y_sh.at[sidx_v.at[0]], bufa_v.at[0], gsem)

    def body(j, carry):
        b = lax.rem(j, 2)
        buf = bufa_v.at[b]
        pltpu.make_async_copy(y_sh.at[sidx_v.at[j]], buf, gsem).wait()
        pltpu.async_copy(y_sh.at[sidx_v.at[j + 1]], bufa_v.at[1 - b], gsem)
        pltpu.sync_copy(buf, acc_sh.at[didx_v.at[j]], add=True)
        return carry

    lax.fori_loop(0, KCH - 1, body, 0)
    last = bufa_v.at[(KCH - 1) % 2]
    pltpu.make_async_copy(y_sh.at[sidx_v.at[KCH - 1]], last, gsem).wait()
    pltpu.sync_copy(last, acc_sh.at[didx_v.at[KCH - 1]], add=True)
    plsc.subcore_barrier()
    pltpu.sync_copy(acc_sh.at[pl.ds(s * RPT, RPT)],
                    out_hbm.at[c, pl.ds(s * RPT, RPT)])


_agg = pl.kernel(
    _agg_body,
    out_type=jax.ShapeDtypeStruct((NCORES, NP, HID), jnp.float32),
    mesh=_sc_mesh,
    compiler_params=_sc_params,
    scratch_types=[
        pltpu.VMEM((KCH, CHUNK), jnp.int32),
        pltpu.VMEM((KCH, CHUNK), jnp.int32),
        pltpu.VMEM((2, CHUNK, HID), jnp.float32),
        pltpu.VMEM_SHARED((NP, HID), jnp.float32),
        pltpu.VMEM_SHARED((NP, HID), jnp.float32),
        pltpu.SemaphoreType.DMA,
    ],
)


def _deg_body(src_hbm, dst_hbm, ones_hbm, zeros_hbm, out_hbm,
              sidx_v, didx_v, ones_v, acc_out, acc_in):
    c = lax.axis_index("c")
    s = lax.axis_index("s")
    wid = c * NTILES + s
    pltpu.sync_copy(zeros_hbm.at[pl.ds(s * RPT, RPT)],
                    acc_out.at[pl.ds(s * RPT, RPT)])
    pltpu.sync_copy(zeros_hbm.at[pl.ds(s * RPT, RPT)],
                    acc_in.at[pl.ds(s * RPT, RPT)])
    pltpu.sync_copy(src_hbm.at[wid], sidx_v)
    pltpu.sync_copy(dst_hbm.at[wid], didx_v)
    pltpu.sync_copy(ones_hbm, ones_v)
    plsc.subcore_barrier()

    def body(j, carry):
        pltpu.sync_copy(ones_v, acc_out.at[sidx_v.at[j]], add=True)
        pltpu.sync_copy(ones_v, acc_in.at[didx_v.at[j]], add=True)
        return carry

    lax.fori_loop(0, DKCH, body, 0)
    plsc.subcore_barrier()
    pltpu.sync_copy(acc_out.at[pl.ds(s * RPT, RPT)],
                    out_hbm.at[c, 0, pl.ds(s * RPT, RPT)])
    pltpu.sync_copy(acc_in.at[pl.ds(s * RPT, RPT)],
                    out_hbm.at[c, 1, pl.ds(s * RPT, RPT)])


_deg = pl.kernel(
    _deg_body,
    out_type=jax.ShapeDtypeStruct((NCORES, 2, NP, 16), jnp.float32),
    mesh=_sc_mesh,
    compiler_params=_sc_params,
    scratch_types=[
        pltpu.VMEM((DKCH, DCH), jnp.int32),
        pltpu.VMEM((DKCH, DCH), jnp.int32),
        pltpu.VMEM((DCH, 16), jnp.float32),
        pltpu.VMEM_SHARED((NP, 16), jnp.float32),
        pltpu.VMEM_SHARED((NP, 16), jnp.float32),
    ],
)


def _l0_body(x_ref, w_ref, d00, d01, d10, d11, m_ref, y_ref, so_ref, si_ref):
    mask = m_ref[...]
    so = lax.rsqrt(jnp.maximum(d00[:, :1] + d01[:, :1], 1.0)) * mask
    si = lax.rsqrt(jnp.maximum(d10[:, :1] + d11[:, :1], 1.0)) * mask
    so_ref[...] = so
    si_ref[...] = si
    y_ref[...] = jnp.dot(x_ref[...] * so, w_ref[...],
                         preferred_element_type=jnp.float32)


_l0 = pl.pallas_call(
    _l0_body,
    grid=(GRID,),
    in_specs=[pl.BlockSpec((RB, 128), lambda i: (i, 0)),
              pl.BlockSpec((128, HID), lambda i: (0, 0)),
              pl.BlockSpec((RB, 16), lambda i: (i, 0)),
              pl.BlockSpec((RB, 16), lambda i: (i, 0)),
              pl.BlockSpec((RB, 16), lambda i: (i, 0)),
              pl.BlockSpec((RB, 16), lambda i: (i, 0)),
              pl.BlockSpec((RB, 1), lambda i: (i, 0))],
    out_specs=(pl.BlockSpec((RB, HID), lambda i: (i, 0)),
               pl.BlockSpec((RB, 1), lambda i: (i, 0)),
               pl.BlockSpec((RB, 1), lambda i: (i, 0))),
    out_shape=(jax.ShapeDtypeStruct((NP, HID), jnp.float32),
               jax.ShapeDtypeStruct((NP, 1), jnp.float32),
               jax.ShapeDtypeStruct((NP, 1), jnp.float32)),
)


def _mid_body(p0_ref, p1_ref, si_ref, b_ref, so_ref, w_ref, o_ref):
    x = jnp.maximum((p0_ref[...] + p1_ref[...]) * si_ref[...] + b_ref[...],
                    0.0) * so_ref[...]
    o_ref[...] = jnp.dot(x, w_ref[...], preferred_element_type=jnp.float32)


_mid = pl.pallas_call(
    _mid_body,
    grid=(GRID,),
    in_specs=[pl.BlockSpec((RB, HID), lambda i: (i, 0)),
              pl.BlockSpec((RB, HID), lambda i: (i, 0)),
              pl.BlockSpec((RB, 1), lambda i: (i, 0)),
              pl.BlockSpec((1, HID), lambda i: (0, 0)),
              pl.BlockSpec((RB, 1), lambda i: (i, 0)),
              pl.BlockSpec((HID, HID), lambda i: (0, 0))],
    out_specs=pl.BlockSpec((RB, HID), lambda i: (i, 0)),
    out_shape=jax.ShapeDtypeStruct((NP, HID), jnp.float32),
)


def _last_body(p0_ref, p1_ref, si_ref, w_ref, b_ref, o_ref):
    x = (p0_ref[...] + p1_ref[...]) * si_ref[...]
    o_ref[...] = jax.nn.sigmoid(
        jnp.dot(x, w_ref[...], preferred_element_type=jnp.float32) + b_ref[...])


_last = pl.pallas_call(
    _last_body,
    grid=(GRID,),
    in_specs=[pl.BlockSpec((RB, HID), lambda i: (i, 0)),
              pl.BlockSpec((RB, HID), lambda i: (i, 0)),
              pl.BlockSpec((RB, 1), lambda i: (i, 0)),
              pl.BlockSpec((HID, 128), lambda i: (0, 0)),
              pl.BlockSpec((1, 128), lambda i: (0, 0))],
    out_specs=pl.BlockSpec((RB, 128), lambda i: (i, 0)),
    out_shape=jax.ShapeDtypeStruct((NP, 128), jnp.float32),
)


def kernel(features, edge_index, Ws, bs):
    pad = jnp.full((EPAD - E,), N, jnp.int32)
    src3 = jnp.concatenate([edge_index[0], pad]).reshape(NW, KCH, CHUNK)
    dst3 = jnp.concatenate([edge_index[1], pad]).reshape(NW, KCH, CHUNK)
    feats_p = jnp.pad(features, ((0, NP - N), (0, 0)))
    zeros64 = jnp.zeros((NP, HID), jnp.float32)
    zeros16 = jnp.zeros((NP, 16), jnp.float32)
    ones16 = jnp.ones((DCH, 16), jnp.float32)
    mask = jnp.pad(jnp.ones((N, 1), jnp.float32), ((0, NP - N), (0, 0)))

    deg = _deg(src3.reshape(NW, DKCH, DCH), dst3.reshape(NW, DKCH, DCH),
               ones16, zeros16)
    y, so, si = _l0(feats_p, Ws[0], deg[0, 0], deg[1, 0], deg[0, 1],
                    deg[1, 1], mask)
    for i in range(1, 7):
        p = _agg(y, src3, dst3, zeros64)
        y = _mid(p[0], p[1], si, bs[i - 1].reshape(1, HID), so, Ws[i])
    p = _agg(y, src3, dst3, zeros64)
    h7 = _mid(p[0], p[1], si, bs[6].reshape(1, HID), so,
              jnp.eye(HID, dtype=jnp.float32))
    q = _agg(h7, src3, dst3, zeros64)
    out = _last(q[0], q[1], si, Ws[7], bs[7].reshape(1, 128))
    return out[:N]


# speedup vs baseline: 2.3778x; 1.0007x over previous
import jax
import jax.numpy as jnp
from jax import lax
from jax.experimental import pallas as pl
from jax.experimental.pallas import tpu as pltpu
from jax.experimental.pallas import tpu_sc as plsc

N = 10000
NP = 10112
E = 320000
NCORES = 2
NTILES = 16
NW = NCORES * NTILES
CHUNK = 128
KCH = 80
DCH = 512
DKCH = 20
EPT = KCH * CHUNK
EPAD = EPT * NW
RPT = NP // NTILES
HID = 64
RB = 2528
GRID = NP // RB

_sc_mesh = plsc.VectorSubcoreMesh(core_axis_name="c", subcore_axis_name="s")
_sc_params = pltpu.CompilerParams(use_tc_tiling_on_sc=False)


def _agg_body(y_hbm, src_hbm, dst_hbm, zeros_hbm, out_hbm,
              sidx_v, didx_v, bufa_v, acc_sh, y_sh, gsem):
    c = lax.axis_index("c")
    s = lax.axis_index("s")
    wid = c * NTILES + s
    pltpu.sync_copy(zeros_hbm.at[pl.ds(s * RPT, RPT)],
                    acc_sh.at[pl.ds(s * RPT, RPT)])
    pltpu.sync_copy(y_hbm.at[pl.ds(s * RPT, RPT)],
                    y_sh.at[pl.ds(s * RPT, RPT)])
    pltpu.sync_copy(src_hbm.at[wid], sidx_v)
    pltpu.sync_copy(dst_hbm.at[wid], didx_v)
    plsc.subcore_barrier()

    pltpu.async_copy(y_sh.at[sidx_v.at[0]], bufa_v.at[0], gsem)

    def body(j, carry):
        b = lax.rem(j, 2)
        buf = bufa_v.at[b]
        pltpu.make_async_copy(y_sh.at[sidx_v.at[j]], buf, gsem).wait()
        pltpu.async_copy(y_sh.at[sidx_v.at[j + 1]], bufa_v.at[1 - b], gsem)
        pltpu.sync_copy(buf, acc_sh.at[didx_v.at[j]], add=True)
        return carry

    lax.fori_loop(0, KCH - 1, body, 0)
    last = bufa_v.at[(KCH - 1) % 2]
    pltpu.make_async_copy(y_sh.at[sidx_v.at[KCH - 1]], last, gsem).wait()
    pltpu.sync_copy(last, acc_sh.at[didx_v.at[KCH - 1]], add=True)
    plsc.subcore_barrier()
    pltpu.sync_copy(acc_sh.at[pl.ds(s * RPT, RPT)],
                    out_hbm.at[c, pl.ds(s * RPT, RPT)])


_agg = pl.kernel(
    _agg_body,
    out_type=jax.ShapeDtypeStruct((NCORES, NP, HID), jnp.float32),
    mesh=_sc_mesh,
    compiler_params=_sc_params,
    scratch_types=[
        pltpu.VMEM((KCH, CHUNK), jnp.int32),
        pltpu.VMEM((KCH, CHUNK), jnp.int32),
        pltpu.VMEM((2, CHUNK, HID), jnp.float32),
        pltpu.VMEM_SHARED((NP, HID), jnp.float32),
        pltpu.VMEM_SHARED((NP, HID), jnp.float32),
        pltpu.SemaphoreType.DMA,
    ],
)


def _deg_body(src_hbm, dst_hbm, ones_hbm, zeros_hbm, out_hbm,
              sidx_v, didx_v, ones_v, acc_out, acc_in, ssem):
    c = lax.axis_index("c")
    s = lax.axis_index("s")
    wid = c * NTILES + s
    pltpu.sync_copy(zeros_hbm.at[pl.ds(s * RPT, RPT)],
                    acc_out.at[pl.ds(s * RPT, RPT)])
    pltpu.sync_copy(zeros_hbm.at[pl.ds(s * RPT, RPT)],
                    acc_in.at[pl.ds(s * RPT, RPT)])
    pltpu.sync_copy(src_hbm.at[wid], sidx_v)
    pltpu.sync_copy(dst_hbm.at[wid], didx_v)
    pltpu.sync_copy(ones_hbm, ones_v)
    plsc.subcore_barrier()

    def body(j, carry):
        pltpu.async_copy(ones_v, acc_out.at[sidx_v.at[j]], ssem, add=True)
        pltpu.async_copy(ones_v, acc_in.at[didx_v.at[j]], ssem, add=True)
        return carry

    lax.fori_loop(0, DKCH, body, 0)

    def drain(j, carry):
        pltpu.make_async_copy(ones_hbm, ones_v, ssem).wait()
        return carry

    lax.fori_loop(0, 2 * DKCH, drain, 0)
    plsc.subcore_barrier()
    pltpu.sync_copy(acc_out.at[pl.ds(s * RPT, RPT)],
                    out_hbm.at[c, 0, pl.ds(s * RPT, RPT)])
    pltpu.sync_copy(acc_in.at[pl.ds(s * RPT, RPT)],
                    out_hbm.at[c, 1, pl.ds(s * RPT, RPT)])


_deg = pl.kernel(
    _deg_body,
    out_type=jax.ShapeDtypeStruct((NCORES, 2, NP, 16), jnp.float32),
    mesh=_sc_mesh,
    compiler_params=_sc_params,
    scratch_types=[
        pltpu.VMEM((DKCH, DCH), jnp.int32),
        pltpu.VMEM((DKCH, DCH), jnp.int32),
        pltpu.VMEM((DCH, 16), jnp.float32),
        pltpu.VMEM_SHARED((NP, 16), jnp.float32),
        pltpu.VMEM_SHARED((NP, 16), jnp.float32),
        pltpu.SemaphoreType.DMA,
    ],
)


def _l0_body(x_ref, w_ref, d00, d01, d10, d11, m_ref, y_ref, so_ref, si_ref):
    mask = m_ref[...]
    so = lax.rsqrt(jnp.maximum(d00[:, :1] + d01[:, :1], 1.0)) * mask
    si = lax.rsqrt(jnp.maximum(d10[:, :1] + d11[:, :1], 1.0)) * mask
    so_ref[...] = so
    si_ref[...] = si
    y_ref[...] = jnp.dot(x_ref[...] * so, w_ref[...],
                         preferred_element_type=jnp.float32)


_l0 = pl.pallas_call(
    _l0_body,
    grid=(GRID,),
    in_specs=[pl.BlockSpec((RB, 128), lambda i: (i, 0)),
              pl.BlockSpec((128, HID), lambda i: (0, 0)),
              pl.BlockSpec((RB, 16), lambda i: (i, 0)),
              pl.BlockSpec((RB, 16), lambda i: (i, 0)),
              pl.BlockSpec((RB, 16), lambda i: (i, 0)),
              pl.BlockSpec((RB, 16), lambda i: (i, 0)),
              pl.BlockSpec((RB, 1), lambda i: (i, 0))],
    out_specs=(pl.BlockSpec((RB, HID), lambda i: (i, 0)),
               pl.BlockSpec((RB, 1), lambda i: (i, 0)),
               pl.BlockSpec((RB, 1), lambda i: (i, 0))),
    out_shape=(jax.ShapeDtypeStruct((NP, HID), jnp.float32),
               jax.ShapeDtypeStruct((NP, 1), jnp.float32),
               jax.ShapeDtypeStruct((NP, 1), jnp.float32)),
)


def _mid_body(p0_ref, p1_ref, si_ref, b_ref, so_ref, w_ref, o_ref):
    x = jnp.maximum((p0_ref[...] + p1_ref[...]) * si_ref[...] + b_ref[...],
                    0.0) * so_ref[...]
    o_ref[...] = jnp.dot(x, w_ref[...], preferred_element_type=jnp.float32)


_mid = pl.pallas_call(
    _mid_body,
    grid=(GRID,),
    in_specs=[pl.BlockSpec((RB, HID), lambda i: (i, 0)),
              pl.BlockSpec((RB, HID), lambda i: (i, 0)),
              pl.BlockSpec((RB, 1), lambda i: (i, 0)),
              pl.BlockSpec((1, HID), lambda i: (0, 0)),
              pl.BlockSpec((RB, 1), lambda i: (i, 0)),
              pl.BlockSpec((HID, HID), lambda i: (0, 0))],
    out_specs=pl.BlockSpec((RB, HID), lambda i: (i, 0)),
    out_shape=jax.ShapeDtypeStruct((NP, HID), jnp.float32),
)


def _last_body(p0_ref, p1_ref, si_ref, w_ref, b_ref, o_ref):
    x = (p0_ref[...] + p1_ref[...]) * si_ref[...]
    o_ref[...] = jax.nn.sigmoid(
        jnp.dot(x, w_ref[...], preferred_element_type=jnp.float32) + b_ref[...])


_last = pl.pallas_call(
    _last_body,
    grid=(GRID,),
    in_specs=[pl.BlockSpec((RB, HID), lambda i: (i, 0)),
              pl.BlockSpec((RB, HID), lambda i: (i, 0)),
              pl.BlockSpec((RB, 1), lambda i: (i, 0)),
              pl.BlockSpec((HID, 128), lambda i: (0, 0)),
              pl.BlockSpec((1, 128), lambda i: (0, 0))],
    out_specs=pl.BlockSpec((RB, 128), lambda i: (i, 0)),
    out_shape=jax.ShapeDtypeStruct((NP, 128), jnp.float32),
)


def kernel(features, edge_index, Ws, bs):
    pad = jnp.full((EPAD - E,), N, jnp.int32)
    src3 = jnp.concatenate([edge_index[0], pad]).reshape(NW, KCH, CHUNK)
    dst3 = jnp.concatenate([edge_index[1], pad]).reshape(NW, KCH, CHUNK)
    feats_p = jnp.pad(features, ((0, NP - N), (0, 0)))
    zeros64 = jnp.zeros((NP, HID), jnp.float32)
    zeros16 = jnp.zeros((NP, 16), jnp.float32)
    ones16 = jnp.ones((DCH, 16), jnp.float32)
    mask = jnp.pad(jnp.ones((N, 1), jnp.float32), ((0, NP - N), (0, 0)))

    deg = _deg(src3.reshape(NW, DKCH, DCH), dst3.reshape(NW, DKCH, DCH),
               ones16, zeros16)
    y, so, si = _l0(feats_p, Ws[0], deg[0, 0], deg[1, 0], deg[0, 1],
                    deg[1, 1], mask)
    for i in range(1, 7):
        p = _agg(y, src3, dst3, zeros64)
        y = _mid(p[0], p[1], si, bs[i - 1].reshape(1, HID), so, Ws[i])
    p = _agg(y, src3, dst3, zeros64)
    h7 = _mid(p[0], p[1], si, bs[6].reshape(1, HID), so,
              jnp.eye(HID, dtype=jnp.float32))
    q = _agg(h7, src3, dst3, zeros64)
    out = _last(q[0], q[1], si, Ws[7], bs[7].reshape(1, 128))
    return out[:N]
